# baseline (speedup 1.0000x reference)
import functools

import jax
import jax.numpy as jnp
from jax import lax
from jax.experimental import pallas as pl
from jax.experimental.pallas import tpu as pltpu
from jax.experimental.pallas import tpu_sc as plsc

N = 10000
DEG = 32
D = 128
OUT = 128

NS = 2560
NT = N - NS
NW = 32
NPW = NS // NW
SC_C = 8
SC_NC = NPW // SC_C
BN = 1240
G = BN // 8
BN2 = 640


def _sc_body(h_hbm, mean_hbm, mn_hbm, mx_hbm, msq_hbm,
             hv0, hv1, mean_v, mn_v, mx_v, msq_v, sem0, sem1):
    wid = lax.axis_index("s") * 2 + lax.axis_index("c")
    base = wid * NPW
    inv = 1.0 / DEG
    hvs = (hv0, hv1)
    sems = (sem0, sem1)

    def copy_for(c, b):
        return pltpu.make_async_copy(
            h_hbm.at[pl.ds(NT + base + c * SC_C, SC_C)], hvs[b], sems[b])

    copy_for(0, 0).start()
    copy_for(1, 1).start()

    def pair_body(cp, carry):
        for b in range(2):
            c = cp * 2 + b
            copy_for(c, b).wait()
            hv = hvs[b]

            def node_body(n, carry2):
                row = c * SC_C + n
                for j in range(D // 16):
                    sl = pl.ds(j * 16, 16)
                    x = hv[n, 0, sl]
                    s = x
                    q = x * x
                    mn = x
                    mx = x
                    for k in range(1, DEG):
                        x = hv[n, k, sl]
                        s = s + x
                        q = q + x * x
                        mn = jnp.minimum(mn, x)
                        mx = jnp.maximum(mx, x)
                    mean_v[row, sl] = s * inv
                    msq_v[row, sl] = q * inv
                    mn_v[row, sl] = mn
                    mx_v[row, sl] = mx
                return carry2

            lax.fori_loop(0, SC_C, node_body, 0)

            @pl.when(c + 2 < SC_NC)
            def _():
                copy_for(c + 2, b).start()
        return carry

    lax.fori_loop(0, SC_NC // 2, pair_body, 0)
    pltpu.sync_copy(mean_v, mean_hbm.at[pl.ds(base, NPW)])
    pltpu.sync_copy(msq_v, msq_hbm.at[pl.ds(base, NPW)])
    pltpu.sync_copy(mn_v, mn_hbm.at[pl.ds(base, NPW)])
    pltpu.sync_copy(mx_v, mx_hbm.at[pl.ds(base, NPW)])


def _sc_aggregate(h):
    mesh = plsc.VectorSubcoreMesh(core_axis_name="c", subcore_axis_name="s")
    f = pl.kernel(
        _sc_body,
        mesh=mesh,
        out_type=[jax.ShapeDtypeStruct((NS, D), jnp.float32)] * 4,
        scratch_types=[
            pltpu.VMEM((SC_C, DEG, D), jnp.float32),
            pltpu.VMEM((SC_C, DEG, D), jnp.float32),
            pltpu.VMEM((NPW, D), jnp.float32),
            pltpu.VMEM((NPW, D), jnp.float32),
            pltpu.VMEM((NPW, D), jnp.float32),
            pltpu.VMEM((NPW, D), jnp.float32),
            pltpu.SemaphoreType.DMA,
            pltpu.SemaphoreType.DMA,
        ],
    )
    return f(h)


def _sublane_reduce8(P, op):
    i2 = jax.lax.broadcasted_iota(jnp.int32, (1, 1, 8, 1), 2)
    m4 = i2 < 4
    m2 = (i2 & 2) == 0
    m1 = (i2 & 1) == 0
    r = op(P, pltpu.roll(P, 4, axis=2))
    m = jnp.where(m4, r[:, 0:4], r[:, 4:8])
    r = op(m, jnp.where(m2, pltpu.roll(m, 6, axis=2),
                        pltpu.roll(m, 2, axis=2)))
    m = jnp.where(m2, r[:, 0:2], r[:, 2:4])
    r = op(m, jnp.where(m1, pltpu.roll(m, 7, axis=2),
                        pltpu.roll(m, 1, axis=2)))
    out = jnp.where(m1, r[:, 0], r[:, 1])
    return out


def _matmul5(mean, mn, mx, std, nf, w, b):
    acc = jnp.dot(mean, w[0:D], preferred_element_type=jnp.float32)
    acc += jnp.dot(mn, w[D:2 * D], preferred_element_type=jnp.float32)
    acc += jnp.dot(mx, w[2 * D:3 * D], preferred_element_type=jnp.float32)
    acc += jnp.dot(std, w[3 * D:4 * D], preferred_element_type=jnp.float32)
    acc += jnp.dot(nf, w[4 * D:5 * D], preferred_element_type=jnp.float32)
    return acc + b


def _pna_kernel(h_ref, nf_ref, w_ref, b_ref, out_ref):
    inv = 1.0 / DEG
    hb = h_ref[...].reshape(BN, DEG // 8, 8, D)
    t0 = hb[:, 0]
    s4 = t0
    q4 = t0 * t0
    mn4 = t0
    mx4 = t0
    for t in range(1, DEG // 8):
        x = hb[:, t]
        s4 = s4 + x
        q4 = q4 + x * x
        mn4 = jnp.minimum(mn4, x)
        mx4 = jnp.maximum(mx4, x)
    add = lambda a, b: a + b
    s = _sublane_reduce8(s4.reshape(G, 8, 8, D), add).reshape(BN, D)
    q = _sublane_reduce8(q4.reshape(G, 8, 8, D), add).reshape(BN, D)
    mn = _sublane_reduce8(mn4.reshape(G, 8, 8, D), jnp.minimum).reshape(BN, D)
    mx = _sublane_reduce8(mx4.reshape(G, 8, 8, D), jnp.maximum).reshape(BN, D)
    mean = s * inv
    var = q * inv - mean * mean
    std = jnp.sqrt(jax.nn.relu(var) + 1e-5)
    out_ref[...] = _matmul5(mean, mn, mx, std, nf_ref[...], w_ref[...],
                            b_ref[...])


def _finish_kernel(mean_ref, mn_ref, mx_ref, msq_ref, nf_ref, w_ref, b_ref,
                   out_ref):
    mean = mean_ref[...]
    var = msq_ref[...] - mean * mean
    std = jnp.sqrt(jax.nn.relu(var) + 1e-5)
    out_ref[...] = _matmul5(mean, mn_ref[...], mx_ref[...], std, nf_ref[...],
                            w_ref[...], b_ref[...])


@jax.jit
def kernel(h, node_feat, W, b):
    b2 = b.reshape(1, OUT)

    mean_sc, mn_sc, mx_sc, msq_sc = _sc_aggregate(h)

    out_tc = pl.pallas_call(
        _pna_kernel,
        grid=(NT // BN,),
        in_specs=[
            pl.BlockSpec((BN, DEG, D), lambda i: (i, 0, 0)),
            pl.BlockSpec((BN, D), lambda i: (i, 0)),
            pl.BlockSpec((5 * D, OUT), lambda i: (0, 0)),
            pl.BlockSpec((1, OUT), lambda i: (0, 0)),
        ],
        out_specs=pl.BlockSpec((BN, OUT), lambda i: (i, 0)),
        out_shape=jax.ShapeDtypeStruct((NT, OUT), jnp.float32),
    )(h, node_feat, W, b2)

    out_sc = pl.pallas_call(
        _finish_kernel,
        grid=(NS // BN2,),
        in_specs=[
            pl.BlockSpec((BN2, D), lambda i: (i, 0)),
            pl.BlockSpec((BN2, D), lambda i: (i, 0)),
            pl.BlockSpec((BN2, D), lambda i: (i, 0)),
            pl.BlockSpec((BN2, D), lambda i: (i, 0)),
            pl.BlockSpec((BN2, D), lambda i: (i, 0)),
            pl.BlockSpec((5 * D, OUT), lambda i: (0, 0)),
            pl.BlockSpec((1, OUT), lambda i: (0, 0)),
        ],
        out_specs=pl.BlockSpec((BN2, OUT), lambda i: (i, 0)),
        out_shape=jax.ShapeDtypeStruct((NS, OUT), jnp.float32),
    )(mean_sc, mn_sc, mx_sc, msq_sc, node_feat[NT:], W, b2)

    return jnp.concatenate([out_tc, out_sc], axis=0)


# speedup vs baseline: 1.0199x; 1.0199x over previous
import functools

import jax
import jax.numpy as jnp
from jax import lax
from jax.experimental import pallas as pl
from jax.experimental.pallas import tpu as pltpu
from jax.experimental.pallas import tpu_sc as plsc

N = 10000
DEG = 32
D = 128
OUT = 128

NS = 3200
NT = N - NS
NW = 32
NPW = NS // NW
SC_C = 5
SC_NC = NPW // SC_C
BN = 400
G = BN // 8
BN2 = 1600


def _sc_body(h_hbm, mean_hbm, mn_hbm, mx_hbm, msq_hbm,
             hv0, hv1, mean_v, mn_v, mx_v, msq_v, sem0, sem1):
    wid = lax.axis_index("s") * 2 + lax.axis_index("c")
    base = wid * NPW
    inv = 1.0 / DEG
    hvs = (hv0, hv1)
    sems = (sem0, sem1)

    def copy_for(c, b):
        return pltpu.make_async_copy(
            h_hbm.at[pl.ds(base + c * SC_C, SC_C)], hvs[b], sems[b])

    copy_for(0, 0).start()
    copy_for(1, 1).start()

    def pair_body(cp, carry):
        for b in range(2):
            c = cp * 2 + b
            copy_for(c, b).wait()
            hv = hvs[b]

            def node_body(n, carry2):
                row = c * SC_C + n
                for j in range(D // 16):
                    sl = pl.ds(j * 16, 16)
                    x = hv[n, 0, sl]
                    s = x
                    q = x * x
                    mn = x
                    mx = x
                    for k in range(1, DEG):
                        x = hv[n, k, sl]
                        s = s + x
                        q = q + x * x
                        mn = jnp.minimum(mn, x)
                        mx = jnp.maximum(mx, x)
                    mean_v[row, sl] = s * inv
                    msq_v[row, sl] = q * inv
                    mn_v[row, sl] = mn
                    mx_v[row, sl] = mx
                return carry2

            lax.fori_loop(0, SC_C, node_body, 0)

            @pl.when(c + 2 < SC_NC)
            def _():
                copy_for(c + 2, b).start()
        return carry

    lax.fori_loop(0, SC_NC // 2, pair_body, 0)
    pltpu.sync_copy(mean_v, mean_hbm.at[wid])
    pltpu.sync_copy(msq_v, msq_hbm.at[wid])
    pltpu.sync_copy(mn_v, mn_hbm.at[wid])
    pltpu.sync_copy(mx_v, mx_hbm.at[wid])


def _sc_aggregate(h):
    mesh = plsc.VectorSubcoreMesh(core_axis_name="c", subcore_axis_name="s")
    f = pl.kernel(
        _sc_body,
        mesh=mesh,
        out_type=[jax.ShapeDtypeStruct((NW, NPW, D), jnp.float32)] * 4,
        scratch_types=[
            pltpu.VMEM((SC_C, DEG, D), jnp.float32),
            pltpu.VMEM((SC_C, DEG, D), jnp.float32),
            pltpu.VMEM((NPW, D), jnp.float32),
            pltpu.VMEM((NPW, D), jnp.float32),
            pltpu.VMEM((NPW, D), jnp.float32),
            pltpu.VMEM((NPW, D), jnp.float32),
            pltpu.SemaphoreType.DMA,
            pltpu.SemaphoreType.DMA,
        ],
    )
    return f(h)


def _sublane_reduce8(P, op):
    i2 = jax.lax.broadcasted_iota(jnp.int32, (1, 1, 8, 1), 2)
    m4 = i2 < 4
    m2 = (i2 & 2) == 0
    m1 = (i2 & 1) == 0
    r = op(P, pltpu.roll(P, 4, axis=2))
    m = jnp.where(m4, r[:, 0:4], r[:, 4:8])
    r = op(m, jnp.where(m2, pltpu.roll(m, 6, axis=2),
                        pltpu.roll(m, 2, axis=2)))
    m = jnp.where(m2, r[:, 0:2], r[:, 2:4])
    r = op(m, jnp.where(m1, pltpu.roll(m, 7, axis=2),
                        pltpu.roll(m, 1, axis=2)))
    out = jnp.where(m1, r[:, 0], r[:, 1])
    return out


def _matmul5(mean, mn, mx, std, nf, w, b):
    acc = jnp.dot(mean, w[0:D], preferred_element_type=jnp.float32)
    acc += jnp.dot(mn, w[D:2 * D], preferred_element_type=jnp.float32)
    acc += jnp.dot(mx, w[2 * D:3 * D], preferred_element_type=jnp.float32)
    acc += jnp.dot(std, w[3 * D:4 * D], preferred_element_type=jnp.float32)
    acc += jnp.dot(nf, w[4 * D:5 * D], preferred_element_type=jnp.float32)
    return acc + b


def _pna_kernel(h_ref, nf_ref, w_ref, b_ref, out_ref):
    inv = 1.0 / DEG
    hb = h_ref[...].reshape(BN, DEG // 8, 8, D)
    t0 = hb[:, 0]
    s4 = t0
    q4 = t0 * t0
    mn4 = t0
    mx4 = t0
    for t in range(1, DEG // 8):
        x = hb[:, t]
        s4 = s4 + x
        q4 = q4 + x * x
        mn4 = jnp.minimum(mn4, x)
        mx4 = jnp.maximum(mx4, x)
    add = lambda a, b: a + b
    s = _sublane_reduce8(s4.reshape(G, 8, 8, D), add).reshape(BN, D)
    q = _sublane_reduce8(q4.reshape(G, 8, 8, D), add).reshape(BN, D)
    mn = _sublane_reduce8(mn4.reshape(G, 8, 8, D), jnp.minimum).reshape(BN, D)
    mx = _sublane_reduce8(mx4.reshape(G, 8, 8, D), jnp.maximum).reshape(BN, D)
    mean = s * inv
    var = q * inv - mean * mean
    std = jnp.sqrt(jax.nn.relu(var) + 1e-5)
    out_ref[...] = _matmul5(mean, mn, mx, std, nf_ref[...], w_ref[...],
                            b_ref[...])


def _finish_kernel(prev_ref, mean_ref, mn_ref, mx_ref, msq_ref, nf_ref,
                   w_ref, b_ref, out_ref):
    del prev_ref
    mean = mean_ref[...]
    var = msq_ref[...] - mean * mean
    std = jnp.sqrt(jax.nn.relu(var) + 1e-5)
    out_ref[...] = _matmul5(mean, mn_ref[...], mx_ref[...], std, nf_ref[...],
                            w_ref[...], b_ref[...])


@jax.jit
def kernel(h, node_feat, W, b):
    b2 = b.reshape(1, OUT)
    off = NS // BN

    mean_sc, mn_sc, mx_sc, msq_sc = _sc_aggregate(h)
    mean_sc = mean_sc.reshape(NS, D)
    mn_sc = mn_sc.reshape(NS, D)
    mx_sc = mx_sc.reshape(NS, D)
    msq_sc = msq_sc.reshape(NS, D)

    out1 = pl.pallas_call(
        _pna_kernel,
        grid=(NT // BN,),
        in_specs=[
            pl.BlockSpec((BN, DEG, D), lambda i: (i + off, 0, 0)),
            pl.BlockSpec((BN, D), lambda i: (i + off, 0)),
            pl.BlockSpec((5 * D, OUT), lambda i: (0, 0)),
            pl.BlockSpec((1, OUT), lambda i: (0, 0)),
        ],
        out_specs=pl.BlockSpec((BN, OUT), lambda i: (i + off, 0)),
        out_shape=jax.ShapeDtypeStruct((N, OUT), jnp.float32),
    )(h, node_feat, W, b2)

    out = pl.pallas_call(
        _finish_kernel,
        grid=(NS // BN2,),
        in_specs=[
            pl.BlockSpec(memory_space=pl.ANY),
            pl.BlockSpec((BN2, D), lambda i: (i, 0)),
            pl.BlockSpec((BN2, D), lambda i: (i, 0)),
            pl.BlockSpec((BN2, D), lambda i: (i, 0)),
            pl.BlockSpec((BN2, D), lambda i: (i, 0)),
            pl.BlockSpec((BN2, D), lambda i: (i, 0)),
            pl.BlockSpec((5 * D, OUT), lambda i: (0, 0)),
            pl.BlockSpec((1, OUT), lambda i: (0, 0)),
        ],
        out_specs=pl.BlockSpec((BN2, OUT), lambda i: (i, 0)),
        out_shape=jax.ShapeDtypeStruct((N, OUT), jnp.float32),
        input_output_aliases={0: 0},
    )(out1, mean_sc, mn_sc, mx_sc, msq_sc, node_feat, W, b2)

    return out


# speedup vs baseline: 1.3454x; 1.3192x over previous
import functools

import jax
import jax.numpy as jnp
from jax.experimental import pallas as pl
from jax.experimental.pallas import tpu as pltpu

N = 10000
DEG = 32
D = 128
OUT = 128
BN = 1000
G = BN // 8


def _sublane_reduce8(P, op):
    i2 = jax.lax.broadcasted_iota(jnp.int32, (1, 1, 8, 1), 2)
    m4 = i2 < 4
    m2 = (i2 & 2) == 0
    m1 = (i2 & 1) == 0
    r = op(P, pltpu.roll(P, 4, axis=2))
    m = jnp.where(m4, r[:, 0:4], r[:, 4:8])
    r = op(m, jnp.where(m2, pltpu.roll(m, 6, axis=2),
                        pltpu.roll(m, 2, axis=2)))
    m = jnp.where(m2, r[:, 0:2], r[:, 2:4])
    r = op(m, jnp.where(m1, pltpu.roll(m, 7, axis=2),
                        pltpu.roll(m, 1, axis=2)))
    out = jnp.where(m1, r[:, 0], r[:, 1])
    return out


def _pna_kernel(h_ref, nf_ref, w_ref, b_ref, out_ref):
    inv = 1.0 / DEG
    hb = h_ref[...].reshape(BN, DEG // 8, 8, D)
    t0 = hb[:, 0]
    s4 = t0
    q4 = t0 * t0
    mn4 = t0
    mx4 = t0
    for t in range(1, DEG // 8):
        x = hb[:, t]
        s4 = s4 + x
        q4 = q4 + x * x
        mn4 = jnp.minimum(mn4, x)
        mx4 = jnp.maximum(mx4, x)
    add = lambda a, b: a + b
    s = _sublane_reduce8(s4.reshape(G, 8, 8, D), add).reshape(BN, D)
    q = _sublane_reduce8(q4.reshape(G, 8, 8, D), add).reshape(BN, D)
    mn = _sublane_reduce8(mn4.reshape(G, 8, 8, D), jnp.minimum).reshape(BN, D)
    mx = _sublane_reduce8(mx4.reshape(G, 8, 8, D), jnp.maximum).reshape(BN, D)
    mean = s * inv
    var = q * inv - mean * mean
    std = jnp.sqrt(jax.nn.relu(var) + 1e-5)
    w = w_ref[...]
    acc = jnp.dot(mean, w[0:D], preferred_element_type=jnp.float32)
    acc += jnp.dot(mn, w[D:2 * D], preferred_element_type=jnp.float32)
    acc += jnp.dot(mx, w[2 * D:3 * D], preferred_element_type=jnp.float32)
    acc += jnp.dot(std, w[3 * D:4 * D], preferred_element_type=jnp.float32)
    acc += jnp.dot(nf_ref[...], w[4 * D:5 * D],
                   preferred_element_type=jnp.float32)
    out_ref[...] = acc + b_ref[...]


@jax.jit
def kernel(h, node_feat, W, b):
    b2 = b.reshape(1, OUT)
    grid = (N // BN,)
    return pl.pallas_call(
        _pna_kernel,
        grid=grid,
        in_specs=[
            pl.BlockSpec((BN, DEG, D), lambda i: (i, 0, 0)),
            pl.BlockSpec((BN, D), lambda i: (i, 0)),
            pl.BlockSpec((5 * D, OUT), lambda i: (0, 0)),
            pl.BlockSpec((1, OUT), lambda i: (0, 0)),
        ],
        out_specs=pl.BlockSpec((BN, OUT), lambda i: (i, 0)),
        out_shape=jax.ShapeDtypeStruct((N, OUT), jnp.float32),
    )(h, node_feat, W, b2)
